# baseline (speedup 1.0000x reference)
import functools

import jax
import jax.numpy as jnp
from jax import lax
from jax.experimental import pallas as pl
from jax.experimental.pallas import tpu as pltpu
from jax.experimental.pallas import tpu_sc as plsc

_NC = 2
_NS = 16
_NW = _NC * _NS
_LANES = 16
_TBL = 256
_NB = 255
_CHUNK = 16384

_SC_SHARE = 9437184
_TC_ROWS = 1024
_TC_COLS = 1024


def _search_vreg(x, tbl_v, cs):
    c31, c63, c95, c127, c159, c191, c223 = cs
    m1 = c127 <= x
    lo = jnp.where(m1, jnp.int32(128), jnp.int32(0))
    t2 = jnp.where(m1, c191, c63)
    m2 = t2 <= x
    lo = lo + jnp.where(m2, jnp.int32(64), jnp.int32(0))
    t3 = jnp.where(m2, jnp.where(m1, c223, c95), jnp.where(m1, c159, c31))
    m3 = t3 <= x
    lo = lo + jnp.where(m3, jnp.int32(32), jnp.int32(0))
    for step in (16, 8, 4, 2, 1):
        g = plsc.load_gather(tbl_v, [lo + (step - 1)])
        lo = lo + jnp.where(g <= x, jnp.int32(step), jnp.int32(0))
    return lo


def _make_sc_call(n):
    per_w = n // _NW
    n_chunks = per_w // _CHUNK
    assert n_chunks % 2 == 0
    mesh = plsc.VectorSubcoreMesh(core_axis_name="c", subcore_axis_name="s")

    @functools.partial(
        pl.kernel,
        out_type=jax.ShapeDtypeStruct((n,), jnp.int32),
        mesh=mesh,
        scratch_types=[
            pltpu.VMEM((_TBL,), jnp.float32),
            pltpu.VMEM((_CHUNK,), jnp.float32),
            pltpu.VMEM((_CHUNK,), jnp.float32),
            pltpu.VMEM((_CHUNK,), jnp.int32),
            pltpu.VMEM((_CHUNK,), jnp.int32),
            pltpu.SemaphoreType.DMA,
            pltpu.SemaphoreType.DMA,
            pltpu.SemaphoreType.DMA,
            pltpu.SemaphoreType.DMA,
        ],
        compiler_params=pltpu.CompilerParams(needs_layout_passes=False),
    )
    def run(in_hbm, tbl_hbm, out_hbm, tbl_v, xa, xb, oa, ob,
            in_sa, in_sb, out_sa, out_sb):
        wid = lax.axis_index("s") * _NC + lax.axis_index("c")
        base = wid * per_w
        pltpu.sync_copy(tbl_hbm, tbl_v)

        cs = tuple(
            plsc.load_gather(tbl_v, [jnp.full((_LANES,), i, jnp.int32)])
            for i in (31, 63, 95, 127, 159, 191, 223)
        )

        def in_copy(ci, buf, sem):
            return pltpu.make_async_copy(
                in_hbm.at[pl.ds(base + ci * _CHUNK, _CHUNK)], buf, sem)

        def out_copy(ci, buf, sem):
            return pltpu.make_async_copy(
                buf, out_hbm.at[pl.ds(base + ci * _CHUNK, _CHUNK)], sem)

        def compute(x_v, o_v):
            @plsc.parallel_loop(0, _CHUNK // _LANES, unroll=8)
            def vec_body(vi):
                x = x_v[pl.ds(vi * _LANES, _LANES)]
                o_v[pl.ds(vi * _LANES, _LANES)] = _search_vreg(x, tbl_v, cs)

        in_copy(0, xa, in_sa).start()
        in_copy(1, xb, in_sb).start()

        def body(ct, carry):
            ca = 2 * ct
            cb = 2 * ct + 1
            in_copy(ca, xa, in_sa).wait()

            @pl.when(ct > 0)
            def _():
                out_copy(ca - 2, oa, out_sa).wait()

            compute(xa, oa)
            out_copy(ca, oa, out_sa).start()

            @pl.when(ct + 1 < n_chunks // 2)
            def _():
                in_copy(ca + 2, xa, in_sa).start()

            in_copy(cb, xb, in_sb).wait()

            @pl.when(ct > 0)
            def _():
                out_copy(cb - 2, ob, out_sb).wait()

            compute(xb, ob)
            out_copy(cb, ob, out_sb).start()

            @pl.when(ct + 1 < n_chunks // 2)
            def _():
                in_copy(cb + 2, xb, in_sb).start()

            return carry

        lax.fori_loop(0, n_chunks // 2, body, 0)
        out_copy(n_chunks - 2, oa, out_sa).wait()
        out_copy(n_chunks - 1, ob, out_sb).wait()

    return run


def _tc_compare_all(x2d, bpad):
    rows, cols = x2d.shape
    grid = rows // _TC_ROWS

    def body(b_s, x_ref, o_ref):
        x = x_ref[...]

        def step(i, acc):
            bi = b_s[i]
            return acc + jnp.where(bi <= x, jnp.int32(1), jnp.int32(0))

        o_ref[...] = lax.fori_loop(
            0, _NB, step, jnp.zeros((_TC_ROWS, _TC_COLS), jnp.int32))

    return pl.pallas_call(
        body,
        grid=(grid,),
        in_specs=[
            pl.BlockSpec(memory_space=pltpu.SMEM),
            pl.BlockSpec((_TC_ROWS, _TC_COLS), lambda i: (i, 0)),
        ],
        out_specs=pl.BlockSpec((_TC_ROWS, _TC_COLS), lambda i: (i, 0)),
        out_shape=jax.ShapeDtypeStruct((rows, cols), jnp.int32),
    )(bpad, x2d)


def kernel(inputs, boundaries):
    n = inputs.shape[0]
    tbl = jnp.concatenate(
        [boundaries, jnp.full((_TBL - _NB,), jnp.inf, dtype=jnp.float32)]
    )
    sc_out = _make_sc_call(_SC_SHARE)(inputs[:_SC_SHARE], tbl)
    tc_n = n - _SC_SHARE
    x2d = inputs[_SC_SHARE:].reshape(tc_n // _TC_COLS, _TC_COLS)
    tc_out = _tc_compare_all(x2d, tbl).reshape(tc_n)
    return jnp.concatenate([sc_out, tc_out])


# speedup vs baseline: 10.7609x; 10.7609x over previous
import functools

import jax
import jax.numpy as jnp
from jax import lax
from jax.experimental import pallas as pl
from jax.experimental.pallas import tpu as pltpu
from jax.experimental.pallas import tpu_sc as plsc

_NC = 2
_NS = 16
_NW = _NC * _NS
_LANES = 16
_TBL = 264
_NB = 255
_CHUNK = 8192
_F = 4
_MININT = -2147483648


def _lut_pallas(bpad):
    def body(b_s, lut_ref):
        k_lo = (lax.broadcasted_iota(jnp.int32, (512, 128), 0) * 128
                + lax.broadcasted_iota(jnp.int32, (512, 128), 1))
        k_hi = k_lo + 65536

        def step(i, accs):
            alo, ahi = accs
            bi = b_s[i] + 0.0
            u = lax.bitcast_convert_type(bi, jnp.int32)
            s = u >> 31
            kb = lax.shift_right_logical(u ^ (s | _MININT), 15)
            one, zero = jnp.int32(1), jnp.int32(0)
            return (alo + jnp.where(kb < k_lo, one, zero),
                    ahi + jnp.where(kb < k_hi, one, zero))

        z = jnp.zeros((512, 128), jnp.int32)
        alo, ahi = lax.fori_loop(0, _NB, step, (z, z))
        lut_ref[...] = alo | (ahi << 16)

    lut = pl.pallas_call(
        body,
        out_shape=jax.ShapeDtypeStruct((512, 128), jnp.int32),
        in_specs=[pl.BlockSpec(memory_space=pltpu.SMEM)],
    )(bpad)
    return lut.reshape(65536)


def _make_sc_call(n):
    per_w = n // _NW
    n_chunks = per_w // _CHUNK
    assert n_chunks % 2 == 0
    mesh = plsc.VectorSubcoreMesh(core_axis_name="c", subcore_axis_name="s")

    @functools.partial(
        pl.kernel,
        out_type=jax.ShapeDtypeStruct((n,), jnp.int32),
        mesh=mesh,
        scratch_types=[
            pltpu.VMEM((_TBL,), jnp.float32),
            pltpu.VMEM((65536,), jnp.int32),
            pltpu.VMEM((_CHUNK,), jnp.float32),
            pltpu.VMEM((_CHUNK,), jnp.float32),
            pltpu.VMEM((_CHUNK,), jnp.int32),
            pltpu.VMEM((_CHUNK,), jnp.int32),
            pltpu.SemaphoreType.DMA,
            pltpu.SemaphoreType.DMA,
            pltpu.SemaphoreType.DMA,
            pltpu.SemaphoreType.DMA,
        ],
        compiler_params=pltpu.CompilerParams(needs_layout_passes=False),
    )
    def run(in_hbm, tbl_hbm, lut_hbm, out_hbm, tbl_v, lut_v, xa, xb, oa, ob,
            in_sa, in_sb, out_sa, out_sb):
        wid = lax.axis_index("s") * _NC + lax.axis_index("c")
        base_off = wid * per_w
        pltpu.sync_copy(tbl_hbm, tbl_v)
        pltpu.sync_copy(lut_hbm, lut_v)

        def in_copy(ci, buf, sem):
            return pltpu.make_async_copy(
                in_hbm.at[pl.ds(base_off + ci * _CHUNK, _CHUNK)], buf, sem)

        def out_copy(ci, buf, sem):
            return pltpu.make_async_copy(
                buf, out_hbm.at[pl.ds(base_off + ci * _CHUNK, _CHUNK)], sem)

        def compute(x_v, o_v):
            one, zero = jnp.int32(1), jnp.int32(0)

            @plsc.parallel_loop(0, _CHUNK // _LANES, unroll=8,
                                carry=jnp.zeros((_LANES,), jnp.int32))
            def vec_body(vi, bad):
                sl = pl.ds(vi * _LANES, _LANES)
                x = x_v[sl] + 0.0
                u = plsc.bitcast(x, jnp.int32)
                s = u >> 31
                key = lax.shift_right_logical(u ^ (s | _MININT), 15)
                wi = key & 65535
                sh = lax.shift_left(lax.shift_right_logical(key, 16), 4)
                w = plsc.load_gather(lut_v, [wi])
                b0 = lax.shift_right_logical(w, sh) & 65535
                g0 = plsc.load_gather(tbl_v, [b0])
                g1 = plsc.load_gather(tbl_v, [b0 + 1])
                g2 = plsc.load_gather(tbl_v, [b0 + 2])
                g3 = plsc.load_gather(tbl_v, [b0 + 3])
                c3 = jnp.where(g3 <= x, one, zero)
                cnt = (jnp.where(g0 <= x, one, zero)
                       + jnp.where(g1 <= x, one, zero)
                       + jnp.where(g2 <= x, one, zero)
                       + c3)
                o_v[sl] = b0 + cnt
                return bad | c3

            bad = vec_body

            @pl.when(jnp.any(bad != 0))
            def _():
                @plsc.parallel_loop(0, _CHUNK // _LANES, unroll=4)
                def redo(vi):
                    sl = pl.ds(vi * _LANES, _LANES)
                    x = x_v[sl]
                    lo = jnp.zeros((_LANES,), jnp.int32)
                    for step in (128, 64, 32, 16, 8, 4, 2, 1):
                        g = plsc.load_gather(tbl_v, [lo + (step - 1)])
                        lo = lo + jnp.where(g <= x, jnp.int32(step), zero)
                    o_v[sl] = lo

        in_copy(0, xa, in_sa).start()
        in_copy(1, xb, in_sb).start()

        def body(ct, carry):
            ca = 2 * ct
            cb = 2 * ct + 1
            in_copy(ca, xa, in_sa).wait()

            @pl.when(ct > 0)
            def _():
                out_copy(ca - 2, oa, out_sa).wait()

            compute(xa, oa)
            out_copy(ca, oa, out_sa).start()

            @pl.when(ct + 1 < n_chunks // 2)
            def _():
                in_copy(ca + 2, xa, in_sa).start()

            in_copy(cb, xb, in_sb).wait()

            @pl.when(ct > 0)
            def _():
                out_copy(cb - 2, ob, out_sb).wait()

            compute(xb, ob)
            out_copy(cb, ob, out_sb).start()

            @pl.when(ct + 1 < n_chunks // 2)
            def _():
                in_copy(cb + 2, xb, in_sb).start()

            return carry

        lax.fori_loop(0, n_chunks // 2, body, 0)
        out_copy(n_chunks - 2, oa, out_sa).wait()
        out_copy(n_chunks - 1, ob, out_sb).wait()

    return run


def kernel(inputs, boundaries):
    n = inputs.shape[0]
    tbl = jnp.concatenate(
        [boundaries, jnp.full((_TBL - _NB,), jnp.inf, dtype=jnp.float32)]
    )
    bpad = jnp.concatenate([boundaries, jnp.zeros((1,), dtype=jnp.float32)])
    lut = _lut_pallas(bpad)
    return _make_sc_call(n)(inputs, tbl, lut)


# speedup vs baseline: 11.3826x; 1.0578x over previous
import functools

import jax
import jax.numpy as jnp
from jax import lax
from jax.experimental import pallas as pl
from jax.experimental.pallas import tpu as pltpu
from jax.experimental.pallas import tpu_sc as plsc

_NC = 2
_NS = 16
_NW = _NC * _NS
_LANES = 16
_TBL = 264
_NB = 255
_CHUNK = 8192
_F = 4
_MININT = -2147483648


def _lut_pallas(bpad):
    def body(b_s, lut_ref):
        k_lo = (lax.broadcasted_iota(jnp.int32, (512, 128), 0) * 128
                + lax.broadcasted_iota(jnp.int32, (512, 128), 1))
        k_hi = k_lo + 65536

        def step(i, accs):
            alo, ahi = accs
            bi = b_s[i] + 0.0
            u = lax.bitcast_convert_type(bi, jnp.int32)
            s = u >> 31
            kb = lax.shift_right_logical(u ^ (s | _MININT), 15)
            one, zero = jnp.int32(1), jnp.int32(0)
            return (alo + jnp.where(kb < k_lo, one, zero),
                    ahi + jnp.where(kb < k_hi, one, zero))

        z = jnp.zeros((512, 128), jnp.int32)
        alo, ahi = lax.fori_loop(0, _NB, step, (z, z))
        lut_ref[...] = alo | (ahi << 16)

    lut = pl.pallas_call(
        body,
        out_shape=jax.ShapeDtypeStruct((512, 128), jnp.int32),
        in_specs=[pl.BlockSpec(memory_space=pltpu.SMEM)],
    )(bpad)
    return lut.reshape(65536)


def _make_sc_call(n):
    per_w = n // _NW
    n_chunks = per_w // _CHUNK
    assert n_chunks % 2 == 0
    mesh = plsc.VectorSubcoreMesh(core_axis_name="c", subcore_axis_name="s")

    @functools.partial(
        pl.kernel,
        out_type=jax.ShapeDtypeStruct((n,), jnp.int32),
        mesh=mesh,
        scratch_types=[
            pltpu.VMEM((_TBL,), jnp.float32),
            pltpu.VMEM((_TBL,), jnp.float32),
            pltpu.VMEM((_TBL,), jnp.float32),
            pltpu.VMEM((_TBL,), jnp.float32),
            pltpu.VMEM((65536,), jnp.int32),
            pltpu.VMEM((_CHUNK,), jnp.float32),
            pltpu.VMEM((_CHUNK,), jnp.float32),
            pltpu.VMEM((_CHUNK,), jnp.int32),
            pltpu.VMEM((_CHUNK,), jnp.int32),
            pltpu.SemaphoreType.DMA,
            pltpu.SemaphoreType.DMA,
            pltpu.SemaphoreType.DMA,
            pltpu.SemaphoreType.DMA,
        ],
        compiler_params=pltpu.CompilerParams(needs_layout_passes=False),
    )
    def run(in_hbm, t0_hbm, t1_hbm, t2_hbm, t3_hbm, lut_hbm, out_hbm,
            tbl_v, t1_v, t2_v, t3_v, lut_v, xa, xb, oa, ob,
            in_sa, in_sb, out_sa, out_sb):
        wid = lax.axis_index("s") * _NC + lax.axis_index("c")
        base_off = wid * per_w
        pltpu.sync_copy(t0_hbm, tbl_v)
        pltpu.sync_copy(t1_hbm, t1_v)
        pltpu.sync_copy(t2_hbm, t2_v)
        pltpu.sync_copy(t3_hbm, t3_v)
        pltpu.sync_copy(lut_hbm, lut_v)

        def in_copy(ci, buf, sem):
            return pltpu.make_async_copy(
                in_hbm.at[pl.ds(base_off + ci * _CHUNK, _CHUNK)], buf, sem)

        def out_copy(ci, buf, sem):
            return pltpu.make_async_copy(
                buf, out_hbm.at[pl.ds(base_off + ci * _CHUNK, _CHUNK)], sem)

        def compute(x_v, o_v):
            one, zero = jnp.int32(1), jnp.int32(0)

            @plsc.parallel_loop(0, _CHUNK // _LANES, unroll=8,
                                carry=jnp.zeros((_LANES,), jnp.int32))
            def vec_body(vi, bad):
                sl = pl.ds(vi * _LANES, _LANES)
                x = x_v[sl] + 0.0
                u = plsc.bitcast(x, jnp.int32)
                s = u >> 31
                key = lax.shift_right_logical(u ^ (s | _MININT), 15)
                wi = key & 65535
                sh = lax.shift_left(lax.shift_right_logical(key, 16), 4)
                w = plsc.load_gather(lut_v, [wi])
                b0 = lax.shift_right_logical(w, sh) & 65535
                g0 = plsc.load_gather(tbl_v, [b0])
                g1 = plsc.load_gather(t1_v, [b0])
                g2 = plsc.load_gather(t2_v, [b0])
                g3 = plsc.load_gather(t3_v, [b0])
                c3 = jnp.where(g3 <= x, one, zero)
                cnt = (jnp.where(g0 <= x, one, zero)
                       + jnp.where(g1 <= x, one, zero)
                       + jnp.where(g2 <= x, one, zero)
                       + c3)
                o_v[sl] = b0 + cnt
                return bad | c3

            bad = vec_body

            @pl.when(jnp.any(bad != 0))
            def _():
                @plsc.parallel_loop(0, _CHUNK // _LANES, unroll=4)
                def redo(vi):
                    sl = pl.ds(vi * _LANES, _LANES)
                    x = x_v[sl]
                    lo = jnp.zeros((_LANES,), jnp.int32)
                    for step in (128, 64, 32, 16, 8, 4, 2, 1):
                        g = plsc.load_gather(tbl_v, [lo + (step - 1)])
                        lo = lo + jnp.where(g <= x, jnp.int32(step), zero)
                    o_v[sl] = lo

        in_copy(0, xa, in_sa).start()
        in_copy(1, xb, in_sb).start()

        def body(ct, carry):
            ca = 2 * ct
            cb = 2 * ct + 1
            in_copy(ca, xa, in_sa).wait()

            @pl.when(ct > 0)
            def _():
                out_copy(ca - 2, oa, out_sa).wait()

            compute(xa, oa)
            out_copy(ca, oa, out_sa).start()

            @pl.when(ct + 1 < n_chunks // 2)
            def _():
                in_copy(ca + 2, xa, in_sa).start()

            in_copy(cb, xb, in_sb).wait()

            @pl.when(ct > 0)
            def _():
                out_copy(cb - 2, ob, out_sb).wait()

            compute(xb, ob)
            out_copy(cb, ob, out_sb).start()

            @pl.when(ct + 1 < n_chunks // 2)
            def _():
                in_copy(cb + 2, xb, in_sb).start()

            return carry

        lax.fori_loop(0, n_chunks // 2, body, 0)
        out_copy(n_chunks - 2, oa, out_sa).wait()
        out_copy(n_chunks - 1, ob, out_sb).wait()

    return run


def kernel(inputs, boundaries):
    n = inputs.shape[0]
    tfull = jnp.concatenate(
        [boundaries, jnp.full((_TBL + 3 - _NB,), jnp.inf, dtype=jnp.float32)]
    )
    t0 = tfull[0:_TBL]
    t1 = tfull[1:_TBL + 1]
    t2 = tfull[2:_TBL + 2]
    t3 = tfull[3:_TBL + 3]
    bpad = jnp.concatenate([boundaries, jnp.zeros((1,), dtype=jnp.float32)])
    lut = _lut_pallas(bpad)
    return _make_sc_call(n)(inputs, t0, t1, t2, t3, lut)
